# Initial kernel scaffold; baseline (speedup 1.0000x reference)
import jax
import jax.numpy as jnp
from jax.experimental import pallas as pl


def kernel(program_ops, emb_table, W_ih, W_hh, b_ih, b_hh):
    raise NotImplementedError("write your pallas kernel here")


# speedup vs baseline: 3.9162x; 3.9162x over previous
import functools

import jax
import jax.numpy as jnp
from jax import lax
from jax.experimental import pallas as pl
from jax.experimental.pallas import tpu as pltpu
from jax.experimental.pallas import tpu_sc as plsc

NUM_OPS = 100000
EMBED_DIM = 64
HIDDEN = 128
B = 4096
T = 50

_NC = 2
_NS = 16
_NW = _NC * _NS
_BT = B * T
_PER_W = _BT // _NW
_CHUNK = 128
_NCHUNK = _PER_W // _CHUNK


def _sc_gather(idx3, table):
    mesh = plsc.VectorSubcoreMesh(core_axis_name="c", subcore_axis_name="s")

    @functools.partial(
        pl.kernel,
        mesh=mesh,
        compiler_params=pltpu.CompilerParams(use_tc_tiling_on_sc=False),
        out_type=jax.ShapeDtypeStruct((_BT, EMBED_DIM), jnp.float32),
        scratch_types=[
            pltpu.VMEM((_NCHUNK, _CHUNK), jnp.int32),
            pltpu.VMEM((_CHUNK, EMBED_DIM), jnp.float32),
            pltpu.VMEM((_CHUNK, EMBED_DIM), jnp.float32),
            pltpu.SemaphoreType.DMA,
            pltpu.SemaphoreType.DMA,
        ],
    )
    def k(idx_hbm, table_hbm, out_hbm, idx_v, buf0, buf1, sem0, sem1):
        wid = lax.axis_index("s") * _NC + lax.axis_index("c")
        base = wid * _PER_W
        pltpu.sync_copy(idx_hbm.at[wid], idx_v)

        bufs = (buf0, buf1)
        sems = (sem0, sem1)

        def start(j, slot):
            pltpu.async_copy(table_hbm.at[idx_v.at[j]], bufs[slot], sems[slot])

        def drain(j, slot):
            pltpu.make_async_copy(
                table_hbm.at[idx_v.at[j]], bufs[slot], sems[slot]
            ).wait()
            pltpu.sync_copy(
                bufs[slot], out_hbm.at[pl.ds(base + j * _CHUNK, _CHUNK)]
            )

        start(0, 0)

        def body(i, _):
            j0 = 2 * i
            start(j0 + 1, 1)
            drain(j0, 0)

            @pl.when(i < _NCHUNK // 2 - 1)
            def _():
                start(j0 + 2, 0)

            drain(j0 + 1, 1)
            return 0

        lax.fori_loop(0, _NCHUNK // 2, body, 0)

    return k(idx3, table)


def _lstm_body(x_ref, w_ref, b_ref, out_ref):
    bblk = x_ref.shape[1]

    def step(t, carry):
        h, c = carry
        xt = x_ref[t]
        cat = jnp.concatenate([xt, h], axis=1)
        g = jnp.dot(cat, w_ref[:], preferred_element_type=jnp.float32) + b_ref[:]
        i = jax.nn.sigmoid(g[:, :HIDDEN])
        f = jax.nn.sigmoid(g[:, HIDDEN:2 * HIDDEN])
        gg = jnp.tanh(g[:, 2 * HIDDEN:3 * HIDDEN])
        o = jax.nn.sigmoid(g[:, 3 * HIDDEN:])
        c2 = f * c + i * gg
        h2 = o * jnp.tanh(c2)
        return (h2, c2)

    h0 = jnp.zeros((bblk, HIDDEN), jnp.float32)
    c0 = jnp.zeros((bblk, HIDDEN), jnp.float32)
    h, _ = lax.fori_loop(0, T, step, (h0, c0))
    out_ref[:] = h


def _lstm(x_tm, w_cat, bias, bblk=512, interpret=False):
    grid = (B // bblk,)
    return pl.pallas_call(
        _lstm_body,
        grid=grid,
        in_specs=[
            pl.BlockSpec((T, bblk, EMBED_DIM), lambda i: (0, i, 0)),
            pl.BlockSpec((EMBED_DIM + HIDDEN, 4 * HIDDEN), lambda i: (0, 0)),
            pl.BlockSpec((1, 4 * HIDDEN), lambda i: (0, 0)),
        ],
        out_specs=pl.BlockSpec((bblk, HIDDEN), lambda i: (i, 0)),
        out_shape=jax.ShapeDtypeStruct((B, HIDDEN), jnp.float32),
        interpret=interpret,
    )(x_tm, w_cat, bias)


def kernel(program_ops, emb_table, W_ih, W_hh, b_ih, b_hh):
    idx3 = jnp.asarray(program_ops, jnp.int32).T.reshape(_NW, _NCHUNK, _CHUNK)
    x = _sc_gather(idx3, emb_table)
    x_tm = x.reshape(T, B, EMBED_DIM)
    w_cat = jnp.concatenate([W_ih.T, W_hh.T], axis=0)
    bias = (b_ih + b_hh).reshape(1, 4 * HIDDEN)
    return _lstm(x_tm, w_cat, bias)


# speedup vs baseline: 3.9924x; 1.0195x over previous
import functools

import jax
import jax.numpy as jnp
from jax import lax
from jax.experimental import pallas as pl
from jax.experimental.pallas import tpu as pltpu
from jax.experimental.pallas import tpu_sc as plsc

NUM_OPS = 100000
EMBED_DIM = 64
HIDDEN = 128
B = 4096
T = 50

_NC = 2
_NS = 16
_NW = _NC * _NS
_BT = B * T
_PER_W = _BT // _NW
_CHUNK = 128
_NCHUNK = _PER_W // _CHUNK


def _sc_gather(idx3, table):
    mesh = plsc.VectorSubcoreMesh(core_axis_name="c", subcore_axis_name="s")

    @functools.partial(
        pl.kernel,
        mesh=mesh,
        compiler_params=pltpu.CompilerParams(use_tc_tiling_on_sc=False),
        out_type=jax.ShapeDtypeStruct((_BT, EMBED_DIM), jnp.bfloat16),
        scratch_types=[
            pltpu.VMEM((_NCHUNK, _CHUNK), jnp.int32),
            pltpu.VMEM((_CHUNK, EMBED_DIM), jnp.bfloat16),
            pltpu.VMEM((_CHUNK, EMBED_DIM), jnp.bfloat16),
            pltpu.SemaphoreType.DMA,
            pltpu.SemaphoreType.DMA,
        ],
    )
    def k(idx_hbm, table_hbm, out_hbm, idx_v, buf0, buf1, sem0, sem1):
        wid = lax.axis_index("s") * _NC + lax.axis_index("c")
        base = wid * _PER_W
        pltpu.sync_copy(idx_hbm.at[wid], idx_v)

        bufs = (buf0, buf1)
        sems = (sem0, sem1)

        def start(j, slot):
            pltpu.async_copy(table_hbm.at[idx_v.at[j]], bufs[slot], sems[slot])

        def drain(j, slot):
            pltpu.make_async_copy(
                table_hbm.at[idx_v.at[j]], bufs[slot], sems[slot]
            ).wait()
            pltpu.sync_copy(
                bufs[slot], out_hbm.at[pl.ds(base + j * _CHUNK, _CHUNK)]
            )

        start(0, 0)

        def body(i, _):
            j0 = 2 * i
            start(j0 + 1, 1)
            drain(j0, 0)

            @pl.when(i < _NCHUNK // 2 - 1)
            def _():
                start(j0 + 2, 0)

            drain(j0 + 1, 1)
            return 0

        lax.fori_loop(0, _NCHUNK // 2, body, 0)

    return k(idx3, table)


_KPAD = 256


def _lstm_body(x_ref, w_ref, out_ref, cat_ref):
    bblk = x_ref.shape[1]

    def sig(v):
        return 0.5 * jnp.tanh(v) + 0.5

    cat_ref[:, EMBED_DIM + HIDDEN:] = jnp.ones(
        (bblk, _KPAD - EMBED_DIM - HIDDEN), jnp.bfloat16)

    half = bblk // 2

    def substep(t, lo, h, c):
        cat_ref[pl.ds(lo, half), :EMBED_DIM] = x_ref[t, pl.ds(lo, half), :]
        cat_ref[pl.ds(lo, half), EMBED_DIM:EMBED_DIM + HIDDEN] = (
            h.astype(jnp.bfloat16))
        g = jnp.dot(cat_ref[pl.ds(lo, half), :], w_ref[:],
                    preferred_element_type=jnp.float32)
        i = sig(g[:, :HIDDEN])
        f = sig(g[:, HIDDEN:2 * HIDDEN])
        gg = jnp.tanh(g[:, 2 * HIDDEN:3 * HIDDEN])
        o = sig(g[:, 3 * HIDDEN:])
        c2 = f * c + i * gg
        h2 = o * jnp.tanh(c2)
        return h2, c2

    def step(t, carry):
        ha, ca, hb, cb = carry
        ha2, ca2 = substep(t, 0, ha, ca)
        hb2, cb2 = substep(t, half, hb, cb)
        return (ha2, ca2, hb2, cb2)

    z = jnp.zeros((half, HIDDEN), jnp.float32)
    ha, _, hb, _ = lax.fori_loop(0, T, step, (z, z, z, z))
    out_ref[:half] = ha
    out_ref[half:] = hb


def _lstm(x_tm, w_aug, bblk=1024, interpret=False):
    grid = (B // bblk,)
    return pl.pallas_call(
        _lstm_body,
        grid=grid,
        in_specs=[
            pl.BlockSpec((T, bblk, EMBED_DIM), lambda i: (0, i, 0)),
            pl.BlockSpec((_KPAD, 4 * HIDDEN), lambda i: (0, 0)),
        ],
        out_specs=pl.BlockSpec((bblk, HIDDEN), lambda i: (i, 0)),
        out_shape=jax.ShapeDtypeStruct((B, HIDDEN), jnp.float32),
        scratch_shapes=[pltpu.VMEM((bblk, _KPAD), jnp.bfloat16)],
        interpret=interpret,
    )(x_tm, w_aug)


def kernel(program_ops, emb_table, W_ih, W_hh, b_ih, b_hh):
    idx3 = jnp.asarray(program_ops, jnp.int32).T.reshape(_NW, _NCHUNK, _CHUNK)
    x = _sc_gather(idx3, emb_table.astype(jnp.bfloat16))
    x_tm = x.reshape(T, B, EMBED_DIM)
    bias = (b_ih + b_hh).reshape(1, 4 * HIDDEN)
    w_aug = jnp.concatenate([
        W_ih.T, W_hh.T, bias,
        jnp.zeros((_KPAD - EMBED_DIM - HIDDEN - 1, 4 * HIDDEN), jnp.float32),
    ], axis=0)
    scale = jnp.concatenate([
        jnp.full((2 * HIDDEN,), 0.5, jnp.float32),
        jnp.ones((HIDDEN,), jnp.float32),
        jnp.full((HIDDEN,), 0.5, jnp.float32),
    ]).reshape(1, 4 * HIDDEN)
    w_aug = (w_aug * scale).astype(jnp.bfloat16)
    return _lstm(x_tm, w_aug)


# speedup vs baseline: 4.4336x; 1.1105x over previous
import functools

import jax
import jax.numpy as jnp
from jax import lax
from jax.experimental import pallas as pl
from jax.experimental.pallas import tpu as pltpu
from jax.experimental.pallas import tpu_sc as plsc

NUM_OPS = 100000
EMBED_DIM = 64
HIDDEN = 128
B = 4096
T = 50

_NC = 2
_NS = 16
_NW = _NC * _NS
_BT = B * T
_PER_W = _BT // _NW
_CHUNK = 128
_NCHUNK = _PER_W // _CHUNK
_E2 = 2 * EMBED_DIM


def _sc_gather(idx2, table):
    mesh = plsc.VectorSubcoreMesh(core_axis_name="c", subcore_axis_name="s")

    @functools.partial(
        pl.kernel,
        mesh=mesh,
        compiler_params=pltpu.CompilerParams(use_tc_tiling_on_sc=False),
        out_type=jax.ShapeDtypeStruct((_BT // 2, _E2), jnp.float32),
        scratch_types=[
            pltpu.VMEM((_CHUNK,), jnp.int32),
            pltpu.VMEM((_CHUNK, EMBED_DIM), jnp.float32),
            pltpu.VMEM((_CHUNK, EMBED_DIM), jnp.float32),
            pltpu.SemaphoreType.DMA,
            pltpu.SemaphoreType.DMA,
        ],
    )
    def k(idx_hbm, table_hbm, out_hbm, idx_v, buf0, buf1, sem0, sem1):
        wid = lax.axis_index("s") * _NC + lax.axis_index("c")
        base = wid * (_PER_W // 2)

        bufs = (buf0, buf1)
        sems = (sem0, sem1)

        def start(j, slot):
            pltpu.sync_copy(idx_hbm.at[wid * _NCHUNK + j], idx_v)
            pltpu.async_copy(table_hbm.at[idx_v], bufs[slot], sems[slot])

        def drain(j, slot):
            pltpu.make_async_copy(table_hbm.at[idx_v], bufs[slot],
                                  sems[slot]).wait()
            r0 = base + j * (_CHUNK // 2)
            pltpu.sync_copy(
                bufs[slot].at[pl.ds(0, _CHUNK // 2)],
                out_hbm.at[pl.ds(r0, _CHUNK // 2), pl.ds(0, EMBED_DIM)],
            )
            pltpu.sync_copy(
                bufs[slot].at[pl.ds(_CHUNK // 2, _CHUNK // 2)],
                out_hbm.at[pl.ds(r0, _CHUNK // 2), pl.ds(EMBED_DIM, EMBED_DIM)],
            )

        start(0, 0)

        def body(i, _):
            j0 = 2 * i
            start(j0 + 1, 1)
            drain(j0, 0)

            @pl.when(i < _NCHUNK // 2 - 1)
            def _():
                start(j0 + 2, 0)

            drain(j0 + 1, 1)
            return 0

        lax.fori_loop(0, _NCHUNK // 2, body, 0)

    return k(idx2, table)


_KP = 256


def _lstm_body(x_ref, w_ref, out_ref, cate_ref, cato_ref):
    half = x_ref.shape[1]
    H = HIDDEN

    def sig(v):
        return 0.5 * jnp.tanh(v) + 0.5

    ones = jnp.ones((half, _KP - EMBED_DIM - H), jnp.bfloat16)
    cate_ref[:, EMBED_DIM + H:] = ones
    cato_ref[:, EMBED_DIM + H:] = ones

    def gates(g, c):
        i = sig(g[:, :H])
        f = sig(g[:, H:2 * H])
        gg = jnp.tanh(g[:, 2 * H:3 * H])
        o = sig(g[:, 3 * H:])
        c2 = f * c + i * gg
        h2 = o * jnp.tanh(c2)
        return h2, c2

    def step(t, carry):
        he, ho, ce, co = carry
        xt = x_ref[t].astype(jnp.bfloat16)
        cate_ref[:, :EMBED_DIM] = xt[:, :EMBED_DIM]
        cate_ref[:, EMBED_DIM:EMBED_DIM + H] = he.astype(jnp.bfloat16)
        cato_ref[:, :EMBED_DIM] = xt[:, EMBED_DIM:]
        cato_ref[:, EMBED_DIM:EMBED_DIM + H] = ho.astype(jnp.bfloat16)
        ge = jnp.dot(cate_ref[:], w_ref[:],
                     preferred_element_type=jnp.float32)
        go = jnp.dot(cato_ref[:], w_ref[:],
                     preferred_element_type=jnp.float32)
        he2, ce2 = gates(ge, ce)
        ho2, co2 = gates(go, co)
        return (he2, ho2, ce2, co2)

    z = jnp.zeros((half, H), jnp.float32)
    he, ho, _, _ = lax.fori_loop(0, T, step, (z, z, z, z))
    out_ref[:, :H] = he
    out_ref[:, H:] = ho


def _lstm(x2, w_aug, bblk=1024, interpret=False):
    half = bblk // 2
    grid = (B // bblk,)
    x3 = x2.reshape(T, B // 2, _E2)
    out = pl.pallas_call(
        _lstm_body,
        grid=grid,
        in_specs=[
            pl.BlockSpec((T, half, _E2), lambda i: (0, i, 0)),
            pl.BlockSpec((_KP, 4 * HIDDEN), lambda i: (0, 0)),
        ],
        out_specs=pl.BlockSpec((half, 2 * HIDDEN), lambda i: (i, 0)),
        out_shape=jax.ShapeDtypeStruct((B // 2, 2 * HIDDEN), jnp.float32),
        scratch_shapes=[
            pltpu.VMEM((half, _KP), jnp.bfloat16),
            pltpu.VMEM((half, _KP), jnp.bfloat16),
        ],
        interpret=interpret,
    )(x3, w_aug)
    return out.reshape(B, HIDDEN)


def _make_w_aug(W_ih, W_hh, b_ih, b_hh):
    H = HIDDEN
    bias = (b_ih + b_hh).reshape(1, 4 * H)
    w = jnp.concatenate([
        W_ih.T, W_hh.T, bias,
        jnp.zeros((_KP - EMBED_DIM - H - 1, 4 * H), jnp.float32),
    ], axis=0)
    scale = jnp.concatenate([
        jnp.full((2 * H,), 0.5, jnp.float32),
        jnp.ones((H,), jnp.float32),
        jnp.full((H,), 0.5, jnp.float32),
    ]).reshape(1, 4 * H)
    return (w * scale).astype(jnp.bfloat16)


def kernel(program_ops, emb_table, W_ih, W_hh, b_ih, b_hh):
    idx2 = (
        jnp.asarray(program_ops, jnp.int32).T
        .reshape(_BT // _CHUNK, _CHUNK // 2, 2)
        .transpose(0, 2, 1)
        .reshape(_BT // _CHUNK, _CHUNK)
    )
    x2 = _sc_gather(idx2, emb_table)
    w_aug = _make_w_aug(W_ih, W_hh, b_ih, b_hh)
    return _lstm(x2, w_aug)


# speedup vs baseline: 5.1823x; 1.1689x over previous
import functools

import jax
import jax.numpy as jnp
from jax import lax
from jax.experimental import pallas as pl
from jax.experimental.pallas import tpu as pltpu
from jax.experimental.pallas import tpu_sc as plsc

NUM_OPS = 100000
EMBED_DIM = 64
HIDDEN = 128
B = 4096
T = 50

_NC = 2
_NS = 16
_NW = _NC * _NS
_BT = B * T
_PER_W = _BT // _NW
_CHUNK = 128
_NCHUNK = _PER_W // _CHUNK
_E2 = 2 * EMBED_DIM


def _sc_gather(idx2, table, nrows):
    mesh = plsc.VectorSubcoreMesh(core_axis_name="c", subcore_axis_name="s")
    per_w = nrows // _NW
    nchunk = per_w // _CHUNK

    @functools.partial(
        pl.kernel,
        mesh=mesh,
        compiler_params=pltpu.CompilerParams(use_tc_tiling_on_sc=False),
        out_type=jax.ShapeDtypeStruct((nrows // 2, _E2), jnp.float32),
        scratch_types=[
            pltpu.VMEM((_CHUNK,), jnp.int32),
            pltpu.VMEM((_CHUNK,), jnp.int32),
            pltpu.VMEM((_CHUNK, EMBED_DIM), jnp.float32),
            pltpu.VMEM((_CHUNK, EMBED_DIM), jnp.float32),
            pltpu.SemaphoreType.DMA,
            pltpu.SemaphoreType.DMA,
        ],
    )
    def k(idx_hbm, table_hbm, out_hbm, idx0, idx1, buf0, buf1, sem0, sem1):
        wid = lax.axis_index("s") * _NC + lax.axis_index("c")
        base = wid * (per_w // 2)

        idxs = (idx0, idx1)
        bufs = (buf0, buf1)
        sems = (sem0, sem1)

        def start(j, slot):
            pltpu.sync_copy(idx_hbm.at[wid * nchunk + j], idxs[slot])
            pltpu.async_copy(table_hbm.at[idxs[slot]], bufs[slot], sems[slot])

        def drain(j, slot):
            pltpu.make_async_copy(table_hbm.at[idxs[slot]], bufs[slot],
                                  sems[slot]).wait()
            r0 = base + j * (_CHUNK // 2)
            pltpu.sync_copy(
                bufs[slot].at[pl.ds(0, _CHUNK // 2)],
                out_hbm.at[pl.ds(r0, _CHUNK // 2), pl.ds(0, EMBED_DIM)],
            )
            pltpu.sync_copy(
                bufs[slot].at[pl.ds(_CHUNK // 2, _CHUNK // 2)],
                out_hbm.at[pl.ds(r0, _CHUNK // 2), pl.ds(EMBED_DIM, EMBED_DIM)],
            )

        start(0, 0)

        def body(i, _):
            @pl.when(i % 2 == 0)
            def _():
                @pl.when(i + 1 < nchunk)
                def _():
                    start(i + 1, 1)
                drain(i, 0)

            @pl.when(i % 2 == 1)
            def _():
                @pl.when(i + 1 < nchunk)
                def _():
                    start(i + 1, 0)
                drain(i, 1)

            return 0

        lax.fori_loop(0, nchunk, body, 0)

    return k(idx2, table)


_KP = 256


def _lstm_body(x_ref, w_ref, out_ref, cate_ref, cato_ref):
    half = x_ref.shape[1]
    H = HIDDEN

    def sig(v):
        return 0.5 * jnp.tanh(v) + 0.5

    ones = jnp.ones((half, _KP - EMBED_DIM - H), jnp.bfloat16)
    cate_ref[:, EMBED_DIM + H:] = ones
    cato_ref[:, EMBED_DIM + H:] = ones

    def gates(g, c):
        i = sig(g[:, :H])
        f = sig(g[:, H:2 * H])
        gg = jnp.tanh(g[:, 2 * H:3 * H])
        o = sig(g[:, 3 * H:])
        c2 = f * c + i * gg
        h2 = o * jnp.tanh(c2)
        return h2, c2

    def step(t, carry):
        he, ho, ce, co = carry
        xt = x_ref[t].astype(jnp.bfloat16)
        cate_ref[:, :EMBED_DIM] = xt[:, :EMBED_DIM]
        cate_ref[:, EMBED_DIM:EMBED_DIM + H] = he.astype(jnp.bfloat16)
        cato_ref[:, :EMBED_DIM] = xt[:, EMBED_DIM:]
        cato_ref[:, EMBED_DIM:EMBED_DIM + H] = ho.astype(jnp.bfloat16)
        ge = jnp.dot(cate_ref[:], w_ref[:],
                     preferred_element_type=jnp.float32)
        go = jnp.dot(cato_ref[:], w_ref[:],
                     preferred_element_type=jnp.float32)
        he2, ce2 = gates(ge, ce)
        ho2, co2 = gates(go, co)
        return (he2, ho2, ce2, co2)

    z = jnp.zeros((half, H), jnp.float32)
    he, ho, _, _ = lax.fori_loop(0, T, step, (z, z, z, z))
    out_ref[:, :H] = he
    out_ref[:, H:] = ho


def _lstm(x2, w_aug, nb=B, bblk=1024, interpret=False):
    half = bblk // 2
    grid = (nb // bblk,)
    x3 = x2.reshape(T, nb // 2, _E2)
    out = pl.pallas_call(
        _lstm_body,
        grid=grid,
        in_specs=[
            pl.BlockSpec((T, half, _E2), lambda i: (0, i, 0)),
            pl.BlockSpec((_KP, 4 * HIDDEN), lambda i: (0, 0)),
        ],
        out_specs=pl.BlockSpec((half, 2 * HIDDEN), lambda i: (i, 0)),
        out_shape=jax.ShapeDtypeStruct((nb // 2, 2 * HIDDEN), jnp.float32),
        scratch_shapes=[
            pltpu.VMEM((half, _KP), jnp.bfloat16),
            pltpu.VMEM((half, _KP), jnp.bfloat16),
        ],
        interpret=interpret,
    )(x3, w_aug)
    return out.reshape(nb, HIDDEN)


def _make_w_aug(W_ih, W_hh, b_ih, b_hh):
    H = HIDDEN
    bias = (b_ih + b_hh).reshape(1, 4 * H)
    w = jnp.concatenate([
        W_ih.T, W_hh.T, bias,
        jnp.zeros((_KP - EMBED_DIM - H - 1, 4 * H), jnp.float32),
    ], axis=0)
    scale = jnp.concatenate([
        jnp.full((2 * H,), 0.5, jnp.float32),
        jnp.ones((H,), jnp.float32),
        jnp.full((H,), 0.5, jnp.float32),
    ]).reshape(1, 4 * H)
    return (w * scale).astype(jnp.bfloat16)


def _mk_idx(po):
    n = po.shape[0] * po.shape[1]
    return (
        po.T.reshape(n // _CHUNK, _CHUNK // 2, 2)
        .transpose(0, 2, 1)
        .reshape(n // _CHUNK, _CHUNK)
    )


def kernel(program_ops, emb_table, W_ih, W_hh, b_ih, b_hh):
    po = jnp.asarray(program_ops, jnp.int32)
    w_aug = _make_w_aug(W_ih, W_hh, b_ih, b_hh)
    hb2 = B // 2
    nr = hb2 * T
    xa = _sc_gather(_mk_idx(po[:hb2]), emb_table, nr)
    xb = _sc_gather(_mk_idx(po[hb2:]), emb_table, nr)
    ha = _lstm(xa, w_aug, nb=hb2)
    hb = _lstm(xb, w_aug, nb=hb2)
    return jnp.concatenate([ha, hb], axis=0)


# speedup vs baseline: 5.2374x; 1.0106x over previous
import functools

import jax
import jax.numpy as jnp
from jax import lax
from jax.experimental import pallas as pl
from jax.experimental.pallas import tpu as pltpu
from jax.experimental.pallas import tpu_sc as plsc

NUM_OPS = 100000
EMBED_DIM = 64
HIDDEN = 128
B = 4096
T = 50

_NC = 2
_NS = 16
_NW = _NC * _NS
_BT = B * T
_PER_W = _BT // _NW
_CHUNK = 128
_NCHUNK = _PER_W // _CHUNK
_E2 = 2 * EMBED_DIM


def _sc_gather(idx2, table, nrows):
    mesh = plsc.VectorSubcoreMesh(core_axis_name="c", subcore_axis_name="s")
    per_w = nrows // _NW
    nchunk = per_w // _CHUNK

    @functools.partial(
        pl.kernel,
        mesh=mesh,
        compiler_params=pltpu.CompilerParams(use_tc_tiling_on_sc=False),
        out_type=jax.ShapeDtypeStruct((nrows // 2, _E2), jnp.float32),
        scratch_types=[
            pltpu.VMEM((_CHUNK,), jnp.int32),
            pltpu.VMEM((_CHUNK,), jnp.int32),
            pltpu.VMEM((_CHUNK, _E2), jnp.float32),
            pltpu.VMEM((_CHUNK, _E2), jnp.float32),
            pltpu.SemaphoreType.DMA,
            pltpu.SemaphoreType.DMA,
        ],
    )
    def k(idx_hbm, table_hbm, out_hbm, idx0, idx1, buf0, buf1, sem0, sem1):
        wid = lax.axis_index("s") * _NC + lax.axis_index("c")
        base = wid * (per_w // 2)

        idxs = (idx0, idx1)
        bufs = (buf0, buf1)
        sems = (sem0, sem1)

        def start(j, slot):
            pltpu.sync_copy(idx_hbm.at[wid * nchunk + j], idxs[slot])
            pltpu.async_copy(table_hbm.at[idxs[slot]], bufs[slot], sems[slot])

        def drain(j, slot):
            pltpu.make_async_copy(table_hbm.at[idxs[slot]], bufs[slot],
                                  sems[slot]).wait()
            r0 = base + j * (_CHUNK // 2)
            pltpu.sync_copy(
                bufs[slot].at[pl.ds(0, _CHUNK // 2), pl.ds(0, EMBED_DIM)],
                out_hbm.at[pl.ds(r0, _CHUNK // 2), pl.ds(0, EMBED_DIM)],
            )
            pltpu.sync_copy(
                bufs[slot].at[pl.ds(_CHUNK // 2, _CHUNK // 2),
                              pl.ds(0, EMBED_DIM)],
                out_hbm.at[pl.ds(r0, _CHUNK // 2), pl.ds(EMBED_DIM, EMBED_DIM)],
            )

        start(0, 0)

        def body(i, _):
            @pl.when(i % 2 == 0)
            def _():
                @pl.when(i + 1 < nchunk)
                def _():
                    start(i + 1, 1)
                drain(i, 0)

            @pl.when(i % 2 == 1)
            def _():
                @pl.when(i + 1 < nchunk)
                def _():
                    start(i + 1, 0)
                drain(i, 1)

            return 0

        lax.fori_loop(0, nchunk, body, 0)

    return k(idx2, table)


_KP = 256


def _lstm_body(x_ref, w_ref, out_ref, cate_ref, cato_ref):
    half = x_ref.shape[1]
    H = HIDDEN

    def sig(v):
        return 0.5 * jnp.tanh(v) + 0.5

    ones = jnp.ones((half, _KP - EMBED_DIM - H), jnp.bfloat16)
    cate_ref[:, EMBED_DIM + H:] = ones
    cato_ref[:, EMBED_DIM + H:] = ones

    def gates(g, c):
        i = sig(g[:, :H])
        f = sig(g[:, H:2 * H])
        gg = jnp.tanh(g[:, 2 * H:3 * H])
        o = sig(g[:, 3 * H:])
        c2 = f * c + i * gg
        h2 = o * jnp.tanh(c2)
        return h2, c2

    def step(t, carry):
        he, ho, ce, co = carry
        xt = x_ref[t].astype(jnp.bfloat16)
        cate_ref[:, :EMBED_DIM] = xt[:, :EMBED_DIM]
        cate_ref[:, EMBED_DIM:EMBED_DIM + H] = he.astype(jnp.bfloat16)
        cato_ref[:, :EMBED_DIM] = xt[:, EMBED_DIM:]
        cato_ref[:, EMBED_DIM:EMBED_DIM + H] = ho.astype(jnp.bfloat16)
        ge = jnp.dot(cate_ref[:], w_ref[:],
                     preferred_element_type=jnp.float32)
        go = jnp.dot(cato_ref[:], w_ref[:],
                     preferred_element_type=jnp.float32)
        he2, ce2 = gates(ge, ce)
        ho2, co2 = gates(go, co)
        return (he2, ho2, ce2, co2)

    z = jnp.zeros((half, H), jnp.float32)
    he, ho, _, _ = lax.fori_loop(0, T, step, (z, z, z, z))
    out_ref[:, :H] = he
    out_ref[:, H:] = ho


def _lstm(x2, w_aug, nb=B, bblk=1024, interpret=False):
    half = bblk // 2
    grid = (nb // bblk,)
    x3 = x2.reshape(T, nb // 2, _E2)
    out = pl.pallas_call(
        _lstm_body,
        grid=grid,
        in_specs=[
            pl.BlockSpec((T, half, _E2), lambda i: (0, i, 0)),
            pl.BlockSpec((_KP, 4 * HIDDEN), lambda i: (0, 0)),
        ],
        out_specs=pl.BlockSpec((half, 2 * HIDDEN), lambda i: (i, 0)),
        out_shape=jax.ShapeDtypeStruct((nb // 2, 2 * HIDDEN), jnp.float32),
        scratch_shapes=[
            pltpu.VMEM((half, _KP), jnp.bfloat16),
            pltpu.VMEM((half, _KP), jnp.bfloat16),
        ],
        interpret=interpret,
    )(x3, w_aug)
    return out.reshape(nb, HIDDEN)


def _make_w_aug(W_ih, W_hh, b_ih, b_hh):
    H = HIDDEN
    bias = (b_ih + b_hh).reshape(1, 4 * H)
    w = jnp.concatenate([
        W_ih.T, W_hh.T, bias,
        jnp.zeros((_KP - EMBED_DIM - H - 1, 4 * H), jnp.float32),
    ], axis=0)
    scale = jnp.concatenate([
        jnp.full((2 * H,), 0.5, jnp.float32),
        jnp.ones((H,), jnp.float32),
        jnp.full((H,), 0.5, jnp.float32),
    ]).reshape(1, 4 * H)
    return (w * scale).astype(jnp.bfloat16)


def _mk_idx(po):
    n = po.shape[0] * po.shape[1]
    return (
        po.T.reshape(n // _CHUNK, _CHUNK // 2, 2)
        .transpose(0, 2, 1)
        .reshape(n // _CHUNK, _CHUNK)
    )


def kernel(program_ops, emb_table, W_ih, W_hh, b_ih, b_hh):
    po = jnp.asarray(program_ops, jnp.int32)
    w_aug = _make_w_aug(W_ih, W_hh, b_ih, b_hh)
    tab = jnp.concatenate(
        [emb_table, jnp.zeros((NUM_OPS, EMBED_DIM), jnp.float32)], axis=1)
    hb2 = B // 2
    nr = hb2 * T
    xa = _sc_gather(_mk_idx(po[:hb2]), tab, nr)
    xb = _sc_gather(_mk_idx(po[hb2:]), tab, nr)
    ha = _lstm(xa, w_aug, nb=hb2)
    hb = _lstm(xb, w_aug, nb=hb2)
    return jnp.concatenate([ha, hb], axis=0)
